# Initial kernel scaffold; baseline (speedup 1.0000x reference)
import jax
import jax.numpy as jnp
from jax.experimental import pallas as pl


def kernel(x, edge_index, adj_matrix, gcn_w, gcn_b, aw_w, aw_b):
    raise NotImplementedError("write your pallas kernel here")


# speedup vs baseline: 187.5345x; 187.5345x over previous
import jax
import jax.numpy as jnp
from jax.experimental import pallas as pl
from jax.experimental.pallas import tpu as pltpu

_BF, _V, _C, _E = 4096, 64, 128, 128
_F = 64


def _build_a_body(ei_ref, a_ref):
    ei = ei_ref[...]
    src = ei[0:1, :]
    dst = ei[1:2, :]
    iota_ve = jax.lax.broadcasted_iota(jnp.int32, (_V, _E), 0)
    s_t = (iota_ve == src).astype(jnp.float32)
    d_t = (iota_ve == dst).astype(jnp.float32)
    deg = jnp.sum(d_t, axis=1, keepdims=True) + 1.0
    dinv = jax.lax.rsqrt(deg)
    dinv_src = jnp.sum(s_t * dinv, axis=0, keepdims=True)
    dinv_dst = jnp.sum(d_t * dinv, axis=0, keepdims=True)
    norm = dinv_src * dinv_dst
    a = jax.lax.dot_general(
        d_t, s_t * norm, (((1,), (1,)), ((), ())),
        preferred_element_type=jnp.float32)
    iota_r = jax.lax.broadcasted_iota(jnp.int32, (_V, _V), 0)
    iota_c = jax.lax.broadcasted_iota(jnp.int32, (_V, _V), 1)
    a = a + jnp.where(iota_r == iota_c, dinv * dinv, 0.0)
    a_ref[...] = a


def _gcn_body(a_ref, w_ref, b_ref, x_ref, o_ref):
    xb = x_ref[...]
    h = jnp.dot(xb.reshape(_F * _V, _C), w_ref[...],
                preferred_element_type=jnp.float32)
    h = h.reshape(_F, _V, _C)
    a_b = jnp.broadcast_to(a_ref[...][None], (_F, _V, _V))
    z = jax.lax.dot_general(
        a_b, h, (((2,), (1,)), ((0,), (0,))),
        preferred_element_type=jnp.float32)
    o_ref[...] = z + b_ref[...][None]


def kernel(x, edge_index, adj_matrix, gcn_w, gcn_b, aw_w, aw_b):
    a = pl.pallas_call(
        _build_a_body,
        out_shape=jax.ShapeDtypeStruct((_V, _V), jnp.float32),
    )(edge_index)

    b2 = gcn_b.reshape(1, _C)
    out = pl.pallas_call(
        _gcn_body,
        grid=(_BF // _F,),
        in_specs=[
            pl.BlockSpec((_V, _V), lambda i: (0, 0)),
            pl.BlockSpec((_C, _C), lambda i: (0, 0)),
            pl.BlockSpec((1, _C), lambda i: (0, 0)),
            pl.BlockSpec((_F, _V, _C), lambda i: (i, 0, 0)),
        ],
        out_specs=pl.BlockSpec((_F, _V, _C), lambda i: (i, 0, 0)),
        out_shape=jax.ShapeDtypeStruct((_BF, _V, _C), jnp.float32),
        compiler_params=pltpu.CompilerParams(
            dimension_semantics=("arbitrary",)),
    )(a, gcn_w, b2, x)
    return out


# speedup vs baseline: 188.3413x; 1.0043x over previous
import jax
import jax.numpy as jnp
from jax.experimental import pallas as pl
from jax.experimental.pallas import tpu as pltpu

_BF, _V, _C, _E = 4096, 64, 128, 128
_F = 64


def _build_a_body(ei_ref, a_ref):
    ei = ei_ref[...]
    src = ei[0:1, :]
    dst = ei[1:2, :]
    iota_ve = jax.lax.broadcasted_iota(jnp.int32, (_V, _E), 0)
    s_t = (iota_ve == src).astype(jnp.float32)
    d_t = (iota_ve == dst).astype(jnp.float32)
    deg = jnp.sum(d_t, axis=1, keepdims=True) + 1.0
    dinv = jax.lax.rsqrt(deg)
    dinv_src = jnp.sum(s_t * dinv, axis=0, keepdims=True)
    dinv_dst = jnp.sum(d_t * dinv, axis=0, keepdims=True)
    norm = dinv_src * dinv_dst
    a = jax.lax.dot_general(
        d_t, s_t * norm, (((1,), (1,)), ((), ())),
        preferred_element_type=jnp.float32)
    iota_r = jax.lax.broadcasted_iota(jnp.int32, (_V, _V), 0)
    iota_c = jax.lax.broadcasted_iota(jnp.int32, (_V, _V), 1)
    a = a + jnp.where(iota_r == iota_c, dinv * dinv, 0.0)
    a_ref[...] = a


def _gcn_body(a_ref, w_ref, b_ref, x_ref, o_ref):
    xb = x_ref[...]
    h = jnp.dot(xb.reshape(_F * _V, _C), w_ref[...],
                preferred_element_type=jnp.float32)
    h = h.reshape(_F, _V, _C)
    a_b = jnp.broadcast_to(a_ref[...][None], (_F, _V, _V))
    z = jax.lax.dot_general(
        a_b, h, (((2,), (1,)), ((0,), (0,))),
        preferred_element_type=jnp.float32)
    o_ref[...] = z + b_ref[...][None]


def kernel(x, edge_index, adj_matrix, gcn_w, gcn_b, aw_w, aw_b):
    a = pl.pallas_call(
        _build_a_body,
        out_shape=jax.ShapeDtypeStruct((_V, _V), jnp.float32),
    )(edge_index)

    b2 = gcn_b.reshape(1, _C)
    out = pl.pallas_call(
        _gcn_body,
        grid=(_BF // _F,),
        in_specs=[
            pl.BlockSpec((_V, _V), lambda i: (0, 0)),
            pl.BlockSpec((_C, _C), lambda i: (0, 0)),
            pl.BlockSpec((1, _C), lambda i: (0, 0)),
            pl.BlockSpec((_F, _V, _C), lambda i: (i, 0, 0)),
        ],
        out_specs=pl.BlockSpec((_F, _V, _C), lambda i: (i, 0, 0)),
        out_shape=jax.ShapeDtypeStruct((_BF, _V, _C), jnp.float32),
        compiler_params=pltpu.CompilerParams(
            dimension_semantics=("parallel",)),
    )(a, gcn_w, b2, x)
    return out


# speedup vs baseline: 229.8728x; 1.2205x over previous
import jax
import jax.numpy as jnp
from jax.experimental import pallas as pl
from jax.experimental.pallas import tpu as pltpu

_BF, _V, _C, _E = 4096, 64, 128, 128
_F = 128


def _build_a_body(ei_ref, a_ref):
    ei = ei_ref[...]
    src = ei[0:1, :]
    dst = ei[1:2, :]
    iota_ve = jax.lax.broadcasted_iota(jnp.int32, (_V, _E), 0)
    s_t = (iota_ve == src).astype(jnp.float32)
    d_t = (iota_ve == dst).astype(jnp.float32)
    deg = jnp.sum(d_t, axis=1, keepdims=True) + 1.0
    dinv = jax.lax.rsqrt(deg)
    dinv_src = jnp.sum(s_t * dinv, axis=0, keepdims=True)
    dinv_dst = jnp.sum(d_t * dinv, axis=0, keepdims=True)
    norm = dinv_src * dinv_dst
    a = jax.lax.dot_general(
        d_t, s_t * norm, (((1,), (1,)), ((), ())),
        preferred_element_type=jnp.float32)
    iota_r = jax.lax.broadcasted_iota(jnp.int32, (_V, _V), 0)
    iota_c = jax.lax.broadcasted_iota(jnp.int32, (_V, _V), 1)
    a = a + jnp.where(iota_r == iota_c, dinv * dinv, 0.0)
    a_ref[...] = a


def _gcn_body(a_ref, w_ref, b_ref, x_ref, o_ref):
    xb = x_ref[...]
    h = jnp.dot(xb.reshape(_F * _V, _C), w_ref[...],
                preferred_element_type=jnp.float32)
    h = h.reshape(_F, _V, _C)
    a_b = jnp.broadcast_to(a_ref[...][None], (_F, _V, _V))
    z = jax.lax.dot_general(
        a_b, h, (((2,), (1,)), ((0,), (0,))),
        preferred_element_type=jnp.float32)
    o_ref[...] = z + b_ref[...][None]


def kernel(x, edge_index, adj_matrix, gcn_w, gcn_b, aw_w, aw_b):
    a = pl.pallas_call(
        _build_a_body,
        out_shape=jax.ShapeDtypeStruct((_V, _V), jnp.float32),
    )(edge_index)

    b2 = gcn_b.reshape(1, _C)
    out = pl.pallas_call(
        _gcn_body,
        grid=(_BF // _F,),
        in_specs=[
            pl.BlockSpec((_V, _V), lambda i: (0, 0)),
            pl.BlockSpec((_C, _C), lambda i: (0, 0)),
            pl.BlockSpec((1, _C), lambda i: (0, 0)),
            pl.BlockSpec((_F, _V, _C), lambda i: (i, 0, 0)),
        ],
        out_specs=pl.BlockSpec((_F, _V, _C), lambda i: (i, 0, 0)),
        out_shape=jax.ShapeDtypeStruct((_BF, _V, _C), jnp.float32),
        compiler_params=pltpu.CompilerParams(
            dimension_semantics=("parallel",)),
    )(a, gcn_w, b2, x)
    return out
